# Initial kernel scaffold; baseline (speedup 1.0000x reference)
import jax
import jax.numpy as jnp
from jax.experimental import pallas as pl


def kernel(x, edge_index, edge_attr, eW, eb, W1, b1, W2, b2, bn_g, bn_b):
    raise NotImplementedError("write your pallas kernel here")


# speedup vs baseline: 2.4654x; 2.4654x over previous
import functools

import jax
import jax.numpy as jnp
from jax import lax
from jax.experimental import pallas as pl
from jax.experimental.pallas import tpu as pltpu
from jax.experimental.pallas import tpu_sc as plsc

N = 10000
E = 320000
HIDDEN = 128
EDGE_DIM = 16
NUM_LAYERS = 3
BN_EPS = 1e-5

NC = 2
NS = 16
LANES = 16

CH = 80
NP = 10240
EB = 4000
BNODE = 1000


def _edge_mlp_body(ea_ref, eW_ref, eb_ref, o0, o1, o2):
    a = ea_ref[...]
    outs = (o0, o1, o2)
    for l in range(NUM_LAYERS):
        r = lax.dot_general(a, eW_ref[l], (((1,), (1,)), ((), ())),
                            preferred_element_type=jnp.float32)
        outs[l][...] = r + eb_ref[l][None, :]


def _edge_mlp(edge_attr, eW, eb):
    nb = E // EB
    return pl.pallas_call(
        _edge_mlp_body,
        grid=(nb,),
        in_specs=[
            pl.BlockSpec((EB, EDGE_DIM), lambda b: (b, 0)),
            pl.BlockSpec((NUM_LAYERS, HIDDEN, EDGE_DIM), lambda b: (0, 0, 0)),
            pl.BlockSpec((NUM_LAYERS, HIDDEN), lambda b: (0, 0)),
        ],
        out_specs=[pl.BlockSpec((EB, HIDDEN), lambda b: (b, 0))] * NUM_LAYERS,
        out_shape=[jax.ShapeDtypeStruct((E, HIDDEN), jnp.float32)] * NUM_LAYERS,
    )(edge_attr, eW, eb)


def _sc_aggr(h, ea_l, src, dst, zeros):
    nw = NC * NS
    epw = E // nw
    nk = epw // CH
    rps = NP // NS
    mesh = plsc.VectorSubcoreMesh(core_axis_name="c", subcore_axis_name="s")

    @functools.partial(
        pl.kernel,
        out_type=jax.ShapeDtypeStruct((NC, NP, HIDDEN), jnp.float32),
        mesh=mesh,
        scratch_types=[
            pltpu.VMEM((CH,), jnp.int32),
            pltpu.VMEM((CH,), jnp.int32),
            pltpu.VMEM((CH, HIDDEN), jnp.float32),
            pltpu.VMEM((CH, HIDDEN), jnp.float32),
            pltpu.VMEM_SHARED((NP, HIDDEN), jnp.float32),
            pltpu.SemaphoreType.DMA,
        ],
    )
    def body(h_hbm, ea_hbm, src_hbm, dst_hbm, z_hbm, out_hbm,
             src_v, dst_v, hrows_v, ea_v, aggr_sh, sem):
        c = lax.axis_index("c")
        s = lax.axis_index("s")
        wid = c * NS + s

        row0 = pl.multiple_of(s * rps, 8)
        pltpu.sync_copy(z_hbm.at[pl.ds(row0, rps)],
                        aggr_sh.at[pl.ds(row0, rps)])
        plsc.subcore_barrier()

        @pl.loop(0, nk)
        def _chunk(k):
            base = wid * epw + k * CH
            pltpu.sync_copy(src_hbm.at[pl.ds(base, CH)], src_v)
            pltpu.sync_copy(dst_hbm.at[pl.ds(base, CH)], dst_v)
            pltpu.async_copy(h_hbm.at[src_v], hrows_v, sem).wait()
            pltpu.sync_copy(ea_hbm.at[pl.ds(base, CH)], ea_v)

            @pl.loop(0, CH)
            def _row(r):
                for j in range(HIDDEN // LANES):
                    sl = pl.ds(j * LANES, LANES)
                    hrows_v[r, sl] = jnp.maximum(hrows_v[r, sl] + ea_v[r, sl],
                                                 0.0)

            pltpu.sync_copy(hrows_v, aggr_sh.at[dst_v], add=True)

        plsc.subcore_barrier()
        pltpu.sync_copy(aggr_sh.at[pl.ds(row0, rps)],
                        out_hbm.at[c, pl.ds(row0, rps)])

    return body(h, ea_l, src, dst, zeros)


def _node_body(h_ref, a_ref, W1_ref, b1_ref, W2_ref, b2_ref, g_ref, bb_ref,
               out_ref, z2_acc, stat_acc):
    ph = pl.program_id(0)
    b = pl.program_id(1)
    nrows = h_ref.shape[0]

    @pl.when(ph == 0)
    def _():
        zin = h_ref[...] + a_ref[0] + a_ref[1]
        t = lax.dot_general(zin, W1_ref[...], (((1,), (1,)), ((), ())),
                            preferred_element_type=jnp.float32) + b1_ref[...]
        t = jnp.maximum(t, 0.0)
        z2 = lax.dot_general(t, W2_ref[...], (((1,), (1,)), ((), ())),
                             preferred_element_type=jnp.float32) + b2_ref[...]
        z2_acc[pl.ds(b * nrows, nrows), :] = z2
        s1 = jnp.sum(z2, axis=0, keepdims=True)
        s2 = jnp.sum(z2 * z2, axis=0, keepdims=True)

        @pl.when(b == 0)
        def _():
            stat_acc[0:1, :] = s1
            stat_acc[1:2, :] = s2

        @pl.when(b != 0)
        def _():
            stat_acc[0:1, :] = stat_acc[0:1, :] + s1
            stat_acc[1:2, :] = stat_acc[1:2, :] + s2

        out_ref[...] = z2

    @pl.when(ph == 1)
    def _():
        mean = stat_acc[0:1, :] / N
        var = stat_acc[1:2, :] / N - mean * mean
        z2 = z2_acc[pl.ds(b * nrows, nrows), :]
        zn = (z2 - mean) * lax.rsqrt(var + BN_EPS) * g_ref[...] + bb_ref[...]
        out_ref[...] = jnp.maximum(zn, 0.0) + 0.1 * h_ref[...]


def _node_mlp(h, a, W1, b1, W2, b2, g, bb):
    nbb = N // BNODE
    vec = pl.BlockSpec((1, HIDDEN), lambda p, b: (0, 0))
    mat = pl.BlockSpec((HIDDEN, HIDDEN), lambda p, b: (0, 0))
    return pl.pallas_call(
        _node_body,
        grid=(2, nbb),
        in_specs=[
            pl.BlockSpec((BNODE, HIDDEN), lambda p, b: (b, 0)),
            pl.BlockSpec((NC, BNODE, HIDDEN), lambda p, b: (0, b, 0)),
            mat, vec, mat, vec, vec, vec,
        ],
        out_specs=pl.BlockSpec((BNODE, HIDDEN), lambda p, b: (b, 0)),
        out_shape=jax.ShapeDtypeStruct((N, HIDDEN), jnp.float32),
        scratch_shapes=[
            pltpu.VMEM((N, HIDDEN), jnp.float32),
            pltpu.VMEM((8, HIDDEN), jnp.float32),
        ],
    )(h, a, W1, b1, W2, b2, g, bb)


def kernel(x, edge_index, edge_attr, eW, eb, W1, b1, W2, b2, bn_g, bn_b):
    src = edge_index[0]
    dst = edge_index[1]
    zeros = jnp.zeros((NP, HIDDEN), jnp.float32)
    eas = _edge_mlp(edge_attr, eW, eb)
    h = x
    for l in range(NUM_LAYERS):
        a = _sc_aggr(h, eas[l], src, dst, zeros)
        h = _node_mlp(h, a, W1[l], b1[l].reshape(1, -1), W2[l],
                      b2[l].reshape(1, -1), bn_g[l].reshape(1, -1),
                      bn_b[l].reshape(1, -1))
    return h


# speedup vs baseline: 2.6604x; 1.0791x over previous
import functools

import jax
import jax.numpy as jnp
from jax import lax
from jax.experimental import pallas as pl
from jax.experimental.pallas import tpu as pltpu
from jax.experimental.pallas import tpu_sc as plsc

N = 10000
E = 320000
HIDDEN = 128
EDGE_DIM = 16
NUM_LAYERS = 3
BN_EPS = 1e-5

NC = 2
NS = 16
LANES = 16

CH = 80
NP = 10240
EB = 4000
BNODE = 1000


def _edge_mlp_body(ea_ref, eW_ref, eb_ref, o0, o1, o2):
    a = ea_ref[...]
    outs = (o0, o1, o2)
    for l in range(NUM_LAYERS):
        r = lax.dot_general(a, eW_ref[l], (((1,), (1,)), ((), ())),
                            preferred_element_type=jnp.float32)
        outs[l][...] = r + eb_ref[l][None, :]


def _edge_mlp(edge_attr, eW, eb):
    nb = E // EB
    return pl.pallas_call(
        _edge_mlp_body,
        grid=(nb,),
        in_specs=[
            pl.BlockSpec((EB, EDGE_DIM), lambda b: (b, 0)),
            pl.BlockSpec((NUM_LAYERS, HIDDEN, EDGE_DIM), lambda b: (0, 0, 0)),
            pl.BlockSpec((NUM_LAYERS, HIDDEN), lambda b: (0, 0)),
        ],
        out_specs=[pl.BlockSpec((EB, HIDDEN), lambda b: (b, 0))] * NUM_LAYERS,
        out_shape=[jax.ShapeDtypeStruct((E, HIDDEN), jnp.float32)] * NUM_LAYERS,
    )(edge_attr, eW, eb)


def _sc_aggr(h, ea_l, ei4, zeros):
    nw = NC * NS
    epw = E // nw
    nk = epw // CH
    rps = NP // NS
    mesh = plsc.VectorSubcoreMesh(core_axis_name="c", subcore_axis_name="s")

    @functools.partial(
        pl.kernel,
        out_type=jax.ShapeDtypeStruct((NC, NP, HIDDEN), jnp.float32),
        mesh=mesh,
        scratch_types=[
            pltpu.VMEM((3, 2, CH), jnp.int32),
            pltpu.VMEM((2, CH, HIDDEN), jnp.float32),
            pltpu.VMEM((2, CH, HIDDEN), jnp.float32),
            pltpu.VMEM_SHARED((NP, HIDDEN), jnp.float32),
            pltpu.SemaphoreType.DMA((3,)),
            pltpu.SemaphoreType.DMA((2,)),
            pltpu.SemaphoreType.DMA((2,)),
            pltpu.SemaphoreType.DMA((2,)),
        ],
    )
    def body(h_hbm, ea_hbm, ei_hbm, z_hbm, out_hbm,
             idx_v, h_v, ea_v, aggr_sh, isem, gsem, esem, ssem):
        c = lax.axis_index("c")
        s = lax.axis_index("s")
        wid = c * NS + s

        row0 = pl.multiple_of(s * rps, 8)
        pltpu.sync_copy(z_hbm.at[pl.ds(row0, rps)],
                        aggr_sh.at[pl.ds(row0, rps)])

        def idx_fetch(k, r):
            pltpu.async_copy(ei_hbm.at[wid, k], idx_v.at[r], isem.at[r])

        def idx_wait(r):
            pltpu.make_async_copy(ei_hbm.at[wid, 0], idx_v.at[r],
                                  isem.at[r]).wait()

        def issue(k, b, r):
            base = pl.multiple_of(wid * epw + k * CH, 8)
            pltpu.async_copy(h_hbm.at[idx_v.at[r, 0]], h_v.at[b], gsem.at[b])
            pltpu.async_copy(ea_hbm.at[pl.ds(base, CH)], ea_v.at[b],
                             esem.at[b])

        idx_fetch(0, 0)
        idx_fetch(1, 1)
        plsc.subcore_barrier()
        idx_wait(0)
        issue(0, 0, 0)

        @pl.loop(0, nk)
        def _chunk(k):
            b = lax.rem(k, 2)
            nb = 1 - b
            r = lax.rem(k, 3)
            rn = lax.rem(k + 1, 3)
            pltpu.make_async_copy(h_hbm.at[idx_v.at[r, 0]], h_v.at[b],
                                  gsem.at[b]).wait()
            pltpu.make_async_copy(ea_hbm.at[pl.ds(0, CH)], ea_v.at[b],
                                  esem.at[b]).wait()

            @pl.when(k > 0)
            def _():
                pltpu.make_async_copy(h_v.at[nb], aggr_sh.at[idx_v.at[r, 1]],
                                      ssem.at[nb]).wait()

            @pl.when(k + 1 < nk)
            def _():
                idx_wait(rn)
                issue(k + 1, nb, rn)

            @pl.when(k + 2 < nk)
            def _():
                idx_fetch(k + 2, lax.rem(k + 2, 3))

            @pl.loop(0, CH, unroll=4)
            def _row(e):
                for j in range(HIDDEN // LANES):
                    sl = pl.ds(j * LANES, LANES)
                    h_v[b, e, sl] = jnp.maximum(h_v[b, e, sl] + ea_v[b, e, sl],
                                                0.0)

            pltpu.async_copy(h_v.at[b], aggr_sh.at[idx_v.at[r, 1]], ssem.at[b],
                             add=True)

        lastb = (nk - 1) % 2
        pltpu.make_async_copy(h_v.at[lastb],
                              aggr_sh.at[idx_v.at[(nk - 1) % 3, 1]],
                              ssem.at[lastb]).wait()
        plsc.subcore_barrier()
        pltpu.sync_copy(aggr_sh.at[pl.ds(row0, rps)],
                        out_hbm.at[c, pl.ds(row0, rps)])

    return body(h, ea_l, ei4, zeros)


def _node_body(h_ref, a_ref, W1_ref, b1_ref, W2_ref, b2_ref, g_ref, bb_ref,
               out_ref, z2_acc, stat_acc):
    ph = pl.program_id(0)
    b = pl.program_id(1)
    nrows = h_ref.shape[0]

    @pl.when(ph == 0)
    def _():
        zin = h_ref[...] + a_ref[0] + a_ref[1]
        t = lax.dot_general(zin, W1_ref[...], (((1,), (1,)), ((), ())),
                            preferred_element_type=jnp.float32) + b1_ref[...]
        t = jnp.maximum(t, 0.0)
        z2 = lax.dot_general(t, W2_ref[...], (((1,), (1,)), ((), ())),
                             preferred_element_type=jnp.float32) + b2_ref[...]
        z2_acc[pl.ds(b * nrows, nrows), :] = z2
        s1 = jnp.sum(z2, axis=0, keepdims=True)
        s2 = jnp.sum(z2 * z2, axis=0, keepdims=True)

        @pl.when(b == 0)
        def _():
            stat_acc[0:1, :] = s1
            stat_acc[1:2, :] = s2

        @pl.when(b != 0)
        def _():
            stat_acc[0:1, :] = stat_acc[0:1, :] + s1
            stat_acc[1:2, :] = stat_acc[1:2, :] + s2

        out_ref[...] = z2

    @pl.when(ph == 1)
    def _():
        mean = stat_acc[0:1, :] / N
        var = stat_acc[1:2, :] / N - mean * mean
        z2 = z2_acc[pl.ds(b * nrows, nrows), :]
        zn = (z2 - mean) * lax.rsqrt(var + BN_EPS) * g_ref[...] + bb_ref[...]
        out_ref[...] = jnp.maximum(zn, 0.0) + 0.1 * h_ref[...]


def _node_mlp(h, a, W1, b1, W2, b2, g, bb):
    nbb = N // BNODE
    vec = pl.BlockSpec((1, HIDDEN), lambda p, b: (0, 0))
    mat = pl.BlockSpec((HIDDEN, HIDDEN), lambda p, b: (0, 0))
    return pl.pallas_call(
        _node_body,
        grid=(2, nbb),
        in_specs=[
            pl.BlockSpec((BNODE, HIDDEN), lambda p, b: (b, 0)),
            pl.BlockSpec((NC, BNODE, HIDDEN), lambda p, b: (0, b, 0)),
            mat, vec, mat, vec, vec, vec,
        ],
        out_specs=pl.BlockSpec((BNODE, HIDDEN), lambda p, b: (b, 0)),
        out_shape=jax.ShapeDtypeStruct((N, HIDDEN), jnp.float32),
        scratch_shapes=[
            pltpu.VMEM((N, HIDDEN), jnp.float32),
            pltpu.VMEM((8, HIDDEN), jnp.float32),
        ],
    )(h, a, W1, b1, W2, b2, g, bb)


def kernel(x, edge_index, edge_attr, eW, eb, W1, b1, W2, b2, bn_g, bn_b):
    nw = NC * NS
    nk = E // nw // CH
    ei4 = jnp.transpose(edge_index.reshape(2, nw, nk, CH), (1, 2, 0, 3))
    zeros = jnp.zeros((NP, HIDDEN), jnp.float32)
    eas = _edge_mlp(edge_attr, eW, eb)
    h = x
    for l in range(NUM_LAYERS):
        a = _sc_aggr(h, eas[l], ei4, zeros)
        h = _node_mlp(h, a, W1[l], b1[l].reshape(1, -1), W2[l],
                      b2[l].reshape(1, -1), bn_g[l].reshape(1, -1),
                      bn_b[l].reshape(1, -1))
    return h


# speedup vs baseline: 4.6687x; 1.7549x over previous
import functools

import jax
import jax.numpy as jnp
from jax import lax
from jax.experimental import pallas as pl
from jax.experimental.pallas import tpu as pltpu
from jax.experimental.pallas import tpu_sc as plsc

N = 10000
E = 320000
HIDDEN = 128
EDGE_DIM = 16
NUM_LAYERS = 3
BN_EPS = 1e-5

NC = 2
NS = 16
LANES = 16

CH = 80
NP = 10240
EB = 4000
BNODE = 1000


def _edge_mlp_body(ea_ref, eW_ref, eb_ref, o0, o1, o2):
    a = ea_ref[...]
    outs = (o0, o1, o2)
    for l in range(NUM_LAYERS):
        r = lax.dot_general(a, eW_ref[l], (((1,), (1,)), ((), ())),
                            preferred_element_type=jnp.float32)
        outs[l][...] = r + eb_ref[l][None, :]


def _edge_mlp(edge_attr, eW, eb):
    nb = E // EB
    return pl.pallas_call(
        _edge_mlp_body,
        grid=(nb,),
        in_specs=[
            pl.BlockSpec((EB, EDGE_DIM), lambda b: (b, 0)),
            pl.BlockSpec((NUM_LAYERS, HIDDEN, EDGE_DIM), lambda b: (0, 0, 0)),
            pl.BlockSpec((NUM_LAYERS, HIDDEN), lambda b: (0, 0)),
        ],
        out_specs=[pl.BlockSpec((EB, HIDDEN), lambda b: (b, 0))] * NUM_LAYERS,
        out_shape=[jax.ShapeDtypeStruct((E, HIDDEN), jnp.float32)] * NUM_LAYERS,
    )(edge_attr, eW, eb)


def _sc_aggr(h, ea_l, ei4, zeros):
    nw = NC * NS
    epw = E // nw
    nk = epw // CH
    rps = NP // NS
    mesh = plsc.VectorSubcoreMesh(core_axis_name="c", subcore_axis_name="s")

    @functools.partial(
        pl.kernel,
        out_type=jax.ShapeDtypeStruct((NC, NP, HIDDEN), jnp.float32),
        mesh=mesh,
        scratch_types=[
            pltpu.VMEM((3, 2, CH), jnp.int32),
            pltpu.VMEM((2, CH, HIDDEN), jnp.float32),
            pltpu.VMEM((2, CH, HIDDEN), jnp.float32),
            pltpu.VMEM_SHARED((NP, HIDDEN), jnp.float32),
            pltpu.SemaphoreType.DMA((3,)),
            pltpu.SemaphoreType.DMA((2,)),
            pltpu.SemaphoreType.DMA((2,)),
            pltpu.SemaphoreType.DMA((2,)),
        ],
    )
    def body(h_hbm, ea_hbm, ei_hbm, z_hbm, out_hbm,
             idx_v, h_v, ea_v, aggr_sh, isem, gsem, esem, ssem):
        c = lax.axis_index("c")
        s = lax.axis_index("s")
        wid = c * NS + s

        row0 = pl.multiple_of(s * rps, 8)
        pltpu.sync_copy(z_hbm.at[pl.ds(row0, rps)],
                        aggr_sh.at[pl.ds(row0, rps)])

        def idx_fetch(k, r):
            pltpu.async_copy(ei_hbm.at[wid, k], idx_v.at[r], isem.at[r])

        def idx_wait(r):
            pltpu.make_async_copy(ei_hbm.at[wid, 0], idx_v.at[r],
                                  isem.at[r]).wait()

        def issue(k, b, r):
            base = pl.multiple_of(wid * epw + k * CH, 8)
            pltpu.async_copy(h_hbm.at[idx_v.at[r, 0]], h_v.at[b], gsem.at[b])
            pltpu.async_copy(ea_hbm.at[pl.ds(base, CH)], ea_v.at[b],
                             esem.at[b])

        idx_fetch(0, 0)
        idx_fetch(1, 1)
        plsc.subcore_barrier()
        idx_wait(0)
        issue(0, 0, 0)

        def chunk_step(k, b):
            nb = 1 - b
            r = lax.rem(k, 3)
            rn = lax.rem(k + 1, 3)
            pltpu.make_async_copy(h_hbm.at[idx_v.at[r, 0]], h_v.at[b],
                                  gsem.at[b]).wait()
            pltpu.make_async_copy(ea_hbm.at[pl.ds(0, CH)], ea_v.at[b],
                                  esem.at[b]).wait()

            @pl.when(k > 0)
            def _():
                pltpu.make_async_copy(h_v.at[nb], aggr_sh.at[idx_v.at[r, 1]],
                                      ssem.at[nb]).wait()

            @pl.when(k + 1 < nk)
            def _():
                idx_wait(rn)
                issue(k + 1, nb, rn)

            @pl.when(k + 2 < nk)
            def _():
                idx_fetch(k + 2, lax.rem(k + 2, 3))

            hb = h_v.at[b]
            eab = ea_v.at[b]

            @plsc.parallel_loop(0, CH, unroll=8)
            def _row(e):
                for j in range(HIDDEN // LANES):
                    sl = pl.ds(j * LANES, LANES)
                    hb[e, sl] = jnp.maximum(hb[e, sl] + eab[e, sl], 0.0)

            pltpu.async_copy(h_v.at[b], aggr_sh.at[idx_v.at[r, 1]], ssem.at[b],
                             add=True)

        @pl.loop(0, nk // 2)
        def _pair(i):
            chunk_step(2 * i, 0)
            chunk_step(2 * i + 1, 1)

        if nk % 2:
            chunk_step(nk - 1, (nk - 1) % 2)

        lastb = (nk - 1) % 2
        pltpu.make_async_copy(h_v.at[lastb],
                              aggr_sh.at[idx_v.at[(nk - 1) % 3, 1]],
                              ssem.at[lastb]).wait()
        plsc.subcore_barrier()
        pltpu.sync_copy(aggr_sh.at[pl.ds(row0, rps)],
                        out_hbm.at[c, pl.ds(row0, rps)])

    return body(h, ea_l, ei4, zeros)


def _node_body(h_ref, a_ref, W1_ref, b1_ref, W2_ref, b2_ref, g_ref, bb_ref,
               out_ref, z2_acc, stat_acc):
    ph = pl.program_id(0)
    b = pl.program_id(1)
    nrows = h_ref.shape[0]

    @pl.when(ph == 0)
    def _():
        zin = h_ref[...] + a_ref[0] + a_ref[1]
        t = lax.dot_general(zin, W1_ref[...], (((1,), (1,)), ((), ())),
                            preferred_element_type=jnp.float32) + b1_ref[...]
        t = jnp.maximum(t, 0.0)
        z2 = lax.dot_general(t, W2_ref[...], (((1,), (1,)), ((), ())),
                             preferred_element_type=jnp.float32) + b2_ref[...]
        z2_acc[pl.ds(b * nrows, nrows), :] = z2
        s1 = jnp.sum(z2, axis=0, keepdims=True)
        s2 = jnp.sum(z2 * z2, axis=0, keepdims=True)

        @pl.when(b == 0)
        def _():
            stat_acc[0:1, :] = s1
            stat_acc[1:2, :] = s2

        @pl.when(b != 0)
        def _():
            stat_acc[0:1, :] = stat_acc[0:1, :] + s1
            stat_acc[1:2, :] = stat_acc[1:2, :] + s2

        out_ref[...] = z2

    @pl.when(ph == 1)
    def _():
        mean = stat_acc[0:1, :] / N
        var = stat_acc[1:2, :] / N - mean * mean
        z2 = z2_acc[pl.ds(b * nrows, nrows), :]
        zn = (z2 - mean) * lax.rsqrt(var + BN_EPS) * g_ref[...] + bb_ref[...]
        out_ref[...] = jnp.maximum(zn, 0.0) + 0.1 * h_ref[...]


def _node_mlp(h, a, W1, b1, W2, b2, g, bb):
    nbb = N // BNODE
    vec = pl.BlockSpec((1, HIDDEN), lambda p, b: (0, 0))
    mat = pl.BlockSpec((HIDDEN, HIDDEN), lambda p, b: (0, 0))
    return pl.pallas_call(
        _node_body,
        grid=(2, nbb),
        in_specs=[
            pl.BlockSpec((BNODE, HIDDEN), lambda p, b: (b, 0)),
            pl.BlockSpec((NC, BNODE, HIDDEN), lambda p, b: (0, b, 0)),
            mat, vec, mat, vec, vec, vec,
        ],
        out_specs=pl.BlockSpec((BNODE, HIDDEN), lambda p, b: (b, 0)),
        out_shape=jax.ShapeDtypeStruct((N, HIDDEN), jnp.float32),
        scratch_shapes=[
            pltpu.VMEM((N, HIDDEN), jnp.float32),
            pltpu.VMEM((8, HIDDEN), jnp.float32),
        ],
    )(h, a, W1, b1, W2, b2, g, bb)


def kernel(x, edge_index, edge_attr, eW, eb, W1, b1, W2, b2, bn_g, bn_b):
    nw = NC * NS
    nk = E // nw // CH
    ei4 = jnp.transpose(edge_index.reshape(2, nw, nk, CH), (1, 2, 0, 3))
    zeros = jnp.zeros((NP, HIDDEN), jnp.float32)
    eas = _edge_mlp(edge_attr, eW, eb)
    h = x
    for l in range(NUM_LAYERS):
        a = _sc_aggr(h, eas[l], ei4, zeros)
        h = _node_mlp(h, a, W1[l], b1[l].reshape(1, -1), W2[l],
                      b2[l].reshape(1, -1), bn_g[l].reshape(1, -1),
                      bn_b[l].reshape(1, -1))
    return h
